# baseline (speedup 1.0000x reference)
import functools

import jax
import jax.numpy as jnp
from jax import lax
from jax.experimental import pallas as pl
from jax.experimental.pallas import tpu as pltpu
from jax.experimental.pallas import tpu_sc as plsc

D = 32
NC = 2
NS = 16
NW = NC * NS
LANES = 128
TB = 4
DP = 33


@functools.lru_cache(maxsize=None)
def _build(T: int):
    n_blk = T // TB
    mesh = plsc.VectorSubcoreMesh(core_axis_name="c", subcore_axis_name="s")

    @functools.partial(
        pl.kernel,
        mesh=mesh,
        out_type=jax.ShapeDtypeStruct((T, D // 8, NW, 8, LANES),
                                      jnp.float32),
        scratch_types=[
            pltpu.VMEM((T // TB, TB * LANES), jnp.int32),
            pltpu.VMEM((2, TB * LANES, DP), jnp.float32),
            pltpu.VMEM((2, TB, D // 8, 1, 8, LANES), jnp.float32),
            pltpu.SemaphoreType.DMA((2,)),
            pltpu.SemaphoreType.DMA((2,)),
        ],
        compiler_params=pltpu.CompilerParams(use_tc_tiling_on_sc=False,
                                             needs_layout_passes=False),
    )
    def gather_kernel(table_hbm, idx_hbm, out_hbm, idx_v, rows_v, tbuf, sg,
                      sw):
        wid = lax.axis_index("s") * NC + lax.axis_index("c")

        pltpu.sync_copy(idx_hbm.at[wid], idx_v)

        def gather_copy(tb, b):
            return pltpu.make_async_copy(
                table_hbm.at[idx_v.at[tb]],
                rows_v.at[b], sg.at[b])

        def wb_copy(tb, b):
            return pltpu.make_async_copy(
                tbuf.at[b],
                out_hbm.at[pl.ds(tb * TB, TB), :, pl.ds(wid, 1)],
                sw.at[b])

        iota = lax.iota(jnp.int32, 16)
        cols = [jnp.full((16,), je, jnp.int32) for je in range(D)]
        gather_copy(0, 0).start()
        gather_copy(1, 1).start()

        def outer(k, carry):
            for d in range(2):
                tb = 2 * k + d
                b = d
                gather_copy(tb, b).wait()

                @pl.when(tb >= 2)
                def _():
                    wb_copy(tb - 2, b).wait()

                rows = rows_v.at[b]
                for tq in range(TB):
                    for v in range(LANES // 16):
                        row = iota + (tq * LANES + 16 * v)
                        for je0 in range(0, D, 8):
                            vecs = [
                                plsc.load_gather(rows, [row, cols[je0 + u]])
                                for u in range(8)
                            ]
                            for u in range(8):
                                je = je0 + u
                                tbuf[b, tq, je // 8, 0, je % 8,
                                     pl.ds(16 * v, 16)] = vecs[u]

                @pl.when(tb + 2 < n_blk)
                def _():
                    gather_copy(tb + 2, b).start()

                wb_copy(tb, b).start()
            return carry

        lax.fori_loop(0, n_blk // 2, outer, 0)
        wb_copy(n_blk - 2, 0).wait()
        wb_copy(n_blk - 1, 1).wait()

    return gather_kernel


def kernel(x, weight):
    Bx, T = x.shape
    xt = x.reshape(NW, LANES, T).transpose(0, 2, 1).reshape(
        NW, T // TB, TB * LANES)
    wp = jnp.pad(weight, ((0, 0), (0, DP - D)))
    out5 = _build(T)(wp, xt)
    return out5.transpose(2, 4, 0, 1, 3).reshape(Bx, T, D)


# speedup vs baseline: 1.5097x; 1.5097x over previous
import functools

import jax
import jax.numpy as jnp
from jax import lax
from jax.experimental import pallas as pl
from jax.experimental.pallas import tpu as pltpu
from jax.experimental.pallas import tpu_sc as plsc

D = 32
NC = 2
NS = 16
NW = NC * NS
LANES = 128
TB = 2
DP = D + 1


@functools.lru_cache(maxsize=None)
def _build(T: int):
    n_blk = T // TB
    mesh = plsc.VectorSubcoreMesh(core_axis_name="c", subcore_axis_name="s")

    @functools.partial(
        pl.kernel,
        mesh=mesh,
        out_type=jax.ShapeDtypeStruct((T, D // 8, NW, 8, LANES),
                                      jnp.float32),
        scratch_types=[
            pltpu.VMEM((T // TB, TB * LANES), jnp.int32),
            pltpu.VMEM((2, TB * LANES, D), jnp.float32),
            pltpu.VMEM((TB * LANES, DP), jnp.float32),
            pltpu.VMEM((2, TB, D // 8, 1, 8, LANES), jnp.float32),
            pltpu.SemaphoreType.DMA((2,)),
            pltpu.SemaphoreType.DMA((2,)),
        ],
        compiler_params=pltpu.CompilerParams(use_tc_tiling_on_sc=False,
                                             needs_layout_passes=False),
    )
    def gather_kernel(table_hbm, idx_hbm, out_hbm, idx_v, rows_v, pad_v,
                      tbuf, sg, sw):
        wid = lax.axis_index("s") * NC + lax.axis_index("c")

        pltpu.sync_copy(idx_hbm.at[wid], idx_v)

        def gather_copy(tb, b):
            return pltpu.make_async_copy(
                table_hbm.at[idx_v.at[tb]],
                rows_v.at[b], sg.at[b])

        def wb_copy(tb, b):
            return pltpu.make_async_copy(
                tbuf.at[b],
                out_hbm.at[pl.ds(tb * TB, TB), :, pl.ds(wid, 1)],
                sw.at[b])

        iota = lax.iota(jnp.int32, 16)
        cols = [jnp.full((16,), je, jnp.int32) for je in range(D)]
        gather_copy(0, 0).start()
        gather_copy(1, 1).start()

        def outer(k, carry):
            for d in range(2):
                tb = 2 * k + d
                b = d
                gather_copy(tb, b).wait()

                @pl.when(tb >= 2)
                def _():
                    wb_copy(tb - 2, b).wait()

                rows = rows_v.at[b]
                for r0 in range(0, TB * LANES, 8):
                    halves = [rows[r0 + q, pl.ds(16 * h, 16)]
                              for q in range(8) for h in range(2)]
                    for q in range(8):
                        for h in range(2):
                            pad_v[r0 + q, pl.ds(16 * h, 16)] = \
                                halves[2 * q + h]
                for tq in range(TB):
                    for v in range(LANES // 16):
                        row = iota + (tq * LANES + 16 * v)
                        for je0 in range(0, D, 8):
                            vecs = [
                                plsc.load_gather(pad_v, [row, cols[je0 + u]])
                                for u in range(8)
                            ]
                            for u in range(8):
                                je = je0 + u
                                tbuf[b, tq, je // 8, 0, je % 8,
                                     pl.ds(16 * v, 16)] = vecs[u]

                @pl.when(tb + 2 < n_blk)
                def _():
                    gather_copy(tb + 2, b).start()

                wb_copy(tb, b).start()
            return carry

        lax.fori_loop(0, n_blk // 2, outer, 0)
        wb_copy(n_blk - 2, 0).wait()
        wb_copy(n_blk - 1, 1).wait()

    return gather_kernel


def kernel(x, weight):
    Bx, T = x.shape
    xt = x.reshape(NW, LANES, T).transpose(0, 2, 1).reshape(
        NW, T // TB, TB * LANES)
    out5 = _build(T)(weight, xt)
    return out5.transpose(2, 4, 0, 1, 3).reshape(Bx, T, D)
